# Initial kernel scaffold; baseline (speedup 1.0000x reference)
import jax
import jax.numpy as jnp
from jax.experimental import pallas as pl


def kernel(x, edge_index, edge_weight, theta_forward, theta_backward):
    raise NotImplementedError("write your pallas kernel here")


# speedup vs baseline: 5.3755x; 5.3755x over previous
import functools

import jax
import jax.numpy as jnp
from jax import lax
from jax.experimental import pallas as pl
from jax.experimental.pallas import tpu as pltpu
from jax.experimental.pallas import tpu_sc as plsc

NC = 2
NS = 16
NW = NC * NS
L = 16

N = 10000
D = 128
FL = (N // NS) // 8 * 8
TAIL = N - NS * FL
NPAD = ((N + NS * L - 1) // (NS * L)) * NS * L
STRIPE = NPAD // NS

E = 320000
EB = ((E // 128 + NW * 8 - 1) // (NW * 8)) * NW * 8
EPAD = EB * 128
RW = EB // NW
R16 = EB // NS

_MESH = plsc.VectorSubcoreMesh(
    core_axis_name="c", subcore_axis_name="s", num_cores=NC, num_subcores=NS)


def _rsqrt_pos(d):
    i = plsc.bitcast(d, jnp.int32)
    i = jnp.int32(0x5F3759DF) - (i >> 1)
    y = plsc.bitcast(i, jnp.float32)
    for _ in range(3):
        y = y * (1.5 - 0.5 * d * y * y)
    return jnp.where(d > 0.0, y, 0.0)


def _norm2_body(row2, col2, w2, nrm_out,
                r_buf, wv_buf, c_buf, deg_buf, dinv_buf, red_buf, nrm_buf,
                sp_deg, sp_dinv):
    c = lax.axis_index("c")
    s = lax.axis_index("s")

    pltpu.sync_copy(row2.at[pl.ds(s * R16, R16)], r_buf)
    pltpu.sync_copy(w2.at[pl.ds(s * R16, R16)], wv_buf)

    def zero(i, carry):
        deg_buf[pl.ds(i * L, L)] = jnp.zeros((L,), jnp.float32)
        return carry
    lax.fori_loop(0, NPAD // L, zero, 0)

    def acc_deg(g, carry):
        gi = g // (128 // L)
        gj = g % (128 // L)
        idx = r_buf[gi, pl.ds(gj * L, L)]
        wv = wv_buf[gi, pl.ds(gj * L, L)]
        plsc.addupdate_scatter(deg_buf, [idx], wv)
        return carry
    lax.fori_loop(0, R16 * (128 // L), acc_deg, 0)

    pltpu.sync_copy(deg_buf, sp_deg.at[s])
    plsc.subcore_barrier()

    pltpu.sync_copy(sp_deg.at[:, pl.ds(s * STRIPE, STRIPE)], red_buf)

    def red(g, carry):
        a = red_buf[0, pl.ds(g * L, L)]
        for r in range(1, NS):
            a = a + red_buf[r, pl.ds(g * L, L)]
        deg_buf[pl.ds(g * L, L)] = _rsqrt_pos(a)
        return carry
    lax.fori_loop(0, STRIPE // L, red, 0)
    pltpu.sync_copy(deg_buf.at[pl.ds(0, STRIPE)],
                    sp_dinv.at[pl.ds(s * STRIPE, STRIPE)])
    plsc.subcore_barrier()

    pltpu.sync_copy(sp_dinv, dinv_buf)
    w = s * NC + c
    pltpu.sync_copy(row2.at[pl.ds(w * RW, RW)], r_buf.at[pl.ds(0, RW)])
    pltpu.sync_copy(col2.at[pl.ds(w * RW, RW)], c_buf)
    pltpu.sync_copy(w2.at[pl.ds(w * RW, RW)], wv_buf.at[pl.ds(0, RW)])

    def nrm(g, carry):
        gi = g // (128 // L)
        gj = g % (128 // L)
        ri = r_buf[gi, pl.ds(gj * L, L)]
        ci = c_buf[gi, pl.ds(gj * L, L)]
        wv = wv_buf[gi, pl.ds(gj * L, L)]
        nv = (plsc.load_gather(dinv_buf, [ri]) * wv
              * plsc.load_gather(dinv_buf, [ci]))
        nrm_buf[gi, pl.ds(gj * L, L)] = nv
        return carry
    lax.fori_loop(0, RW * (128 // L), nrm, 0)
    pltpu.sync_copy(nrm_buf, nrm_out.at[pl.ds(w * RW, RW)])


_norm2_kernel = functools.partial(
    pl.kernel,
    out_type=jax.ShapeDtypeStruct((EB, 128), jnp.float32),
    mesh=_MESH,
    compiler_params=pltpu.CompilerParams(needs_layout_passes=False),
    scratch_types=[
        pltpu.VMEM((R16, 128), jnp.int32),
        pltpu.VMEM((R16, 128), jnp.float32),
        pltpu.VMEM((RW, 128), jnp.int32),
        pltpu.VMEM((NPAD,), jnp.float32),
        pltpu.VMEM((NPAD,), jnp.float32),
        pltpu.VMEM((NS, STRIPE), jnp.float32),
        pltpu.VMEM((RW, 128), jnp.float32),
        pltpu.VMEM_SHARED((NS, NPAD), jnp.float32),
        pltpu.VMEM_SHARED((NPAD,), jnp.float32),
    ],
)(_norm2_body)


def _hop_body(x_hbm, row2, col2, nrm2, zeros_hbm, out_hbm,
              row_t, col_t, nrm_t, xrows, acc, sem):
    c = lax.axis_index("c")
    s = lax.axis_index("s")
    w = s * NC + c

    pltpu.sync_copy(row2.at[pl.ds(w * RW, RW)], row_t)
    pltpu.sync_copy(col2.at[pl.ds(w * RW, RW)], col_t)
    pltpu.sync_copy(nrm2.at[pl.ds(w * RW, RW)], nrm_t)
    pltpu.sync_copy(zeros_hbm.at[pl.ds(0, FL)], acc.at[pl.ds(s * FL, FL)])

    @pl.when(s == NS - 1)
    def _zero_tail():
        pltpu.sync_copy(zeros_hbm.at[pl.ds(FL, TAIL)],
                        acc.at[pl.ds(NS * FL, TAIL)])
    plsc.subcore_barrier()

    def batch(i, carry):
        pltpu.async_copy(x_hbm.at[col_t.at[i]], xrows, sem).wait()

        def scale(e, c2):
            sv = plsc.load_gather(
                nrm_t, [jnp.full((L,), i, jnp.int32),
                        jnp.full((L,), e, jnp.int32)])
            for j in range(D // L):
                xrows[e, pl.ds(j * L, L)] = xrows[e, pl.ds(j * L, L)] * sv
            return c2
        lax.fori_loop(0, 128, scale, 0)
        pltpu.sync_copy(xrows, acc.at[row_t.at[i]], add=True)
        return carry
    lax.fori_loop(0, RW, batch, 0)

    plsc.subcore_barrier()
    pltpu.sync_copy(acc.at[pl.ds(s * FL, FL)],
                    out_hbm.at[c, pl.ds(s * FL, FL)])

    @pl.when(s == NS - 1)
    def _flush_tail():
        pltpu.sync_copy(acc.at[pl.ds(NS * FL, TAIL)],
                        out_hbm.at[c, pl.ds(NS * FL, TAIL)])


_hop_kernel = functools.partial(
    pl.kernel,
    out_type=jax.ShapeDtypeStruct((NC, N, D), jnp.float32),
    mesh=_MESH,
    compiler_params=pltpu.CompilerParams(needs_layout_passes=False),
    scratch_types=[
        pltpu.VMEM((RW, 128), jnp.int32),
        pltpu.VMEM((RW, 128), jnp.int32),
        pltpu.VMEM((RW, 128), jnp.float32),
        pltpu.VMEM((128, D), jnp.float32),
        pltpu.VMEM_SHARED((N, D), jnp.float32),
        pltpu.SemaphoreType.DMA,
    ],
)(_hop_body)


def _combine_body(p_ref, o_ref):
    o_ref[...] = p_ref[0] + p_ref[1]


def _combine(p):
    nb = 10
    return pl.pallas_call(
        _combine_body,
        grid=(nb,),
        in_specs=[pl.BlockSpec((NC, N // nb, D), lambda i: (0, i, 0))],
        out_specs=pl.BlockSpec((N // nb, D), lambda i: (i, 0)),
        out_shape=jax.ShapeDtypeStruct((N, D), jnp.float32),
    )(p)


def _mm_body(x_ref, h1_ref, h2_ref, h3p_ref, tf_ref, tb_ref, o_ref):
    w0 = tf_ref[0]
    w1 = tb_ref[0] + tb_ref[1]
    w2 = tf_ref[1] + tb_ref[2]
    w3 = tf_ref[2]
    h3 = h3p_ref[0] + h3p_ref[1]
    o_ref[...] = (
        jnp.dot(x_ref[...], w0, preferred_element_type=jnp.float32)
        + jnp.dot(h1_ref[...], w1, preferred_element_type=jnp.float32)
        + jnp.dot(h2_ref[...], w2, preferred_element_type=jnp.float32)
        + jnp.dot(h3, w3, preferred_element_type=jnp.float32))


def _mm(x, h1, h2, h3p, tf, tb):
    nb = 10
    blk = N // nb
    return pl.pallas_call(
        _mm_body,
        grid=(nb,),
        in_specs=[
            pl.BlockSpec((blk, D), lambda i: (i, 0)),
            pl.BlockSpec((blk, D), lambda i: (i, 0)),
            pl.BlockSpec((blk, D), lambda i: (i, 0)),
            pl.BlockSpec((NC, blk, D), lambda i: (0, i, 0)),
            pl.BlockSpec((3, D, D), lambda i: (0, 0, 0)),
            pl.BlockSpec((3, D, D), lambda i: (0, 0, 0)),
        ],
        out_specs=pl.BlockSpec((blk, D), lambda i: (i, 0)),
        out_shape=jax.ShapeDtypeStruct((N, D), jnp.float32),
    )(x, h1, h2, h3p, tf, tb)


def kernel(x, edge_index, edge_weight, theta_forward, theta_backward):
    row = edge_index[0]
    col = edge_index[1]
    pad = EPAD - row.shape[0]
    row2 = jnp.concatenate(
        [row, jnp.zeros((pad,), jnp.int32)]).reshape(EB, 128)
    col2 = jnp.concatenate(
        [col, jnp.zeros((pad,), jnp.int32)]).reshape(EB, 128)
    w2 = jnp.concatenate(
        [edge_weight, jnp.zeros((pad,), jnp.float32)]).reshape(EB, 128)

    nrm2 = _norm2_kernel(row2, col2, w2)
    zeros = jnp.zeros((FL + TAIL, D), jnp.float32)

    h1p = _hop_kernel(x, row2, col2, nrm2, zeros)
    h1 = _combine(h1p)
    h2p = _hop_kernel(h1, row2, col2, nrm2, zeros)
    h2 = _combine(h2p)
    h3p = _hop_kernel(h2, row2, col2, nrm2, zeros)
    return _mm(x, h1, h2, h3p, theta_forward, theta_backward)
